# baseline (speedup 1.0000x reference)
import jax
import jax.numpy as jnp
from jax.experimental import pallas as pl
from jax.experimental.pallas import tpu as pltpu

DIM = 1024
FFN = 4096
E = 8
N = 4096

BM = 512
BK = 1024
KF = FFN // BK
NBLK = N // BM + E
CAP = NBLK * BM

_INV_SQRT2 = 0.7071067811865476


def _gated_mlp_block(be_ref, x_ref, w1_ref, b1_ref, w2_ref, b2_ref,
                     wg_ref, bg_ref, out_ref, acc_ref):
    k = pl.program_id(1)
    xb = x_ref[...]
    xbf = xb.astype(jnp.bfloat16)
    h = jnp.dot(xbf, w1_ref[0].astype(jnp.bfloat16),
                preferred_element_type=jnp.float32)
    h = h + b1_ref[0]
    h = 0.5 * h * (1.0 + jax.lax.erf(h * _INV_SQRT2))
    t = jnp.dot(h.astype(jnp.bfloat16), w2_ref[0].astype(jnp.bfloat16),
                preferred_element_type=jnp.float32)

    @pl.when(k == 0)
    def _():
        acc_ref[...] = t

    @pl.when(k > 0)
    def _():
        acc_ref[...] = acc_ref[...] + t

    @pl.when(k == KF - 1)
    def _():
        tt = acc_ref[...] + b2_ref[0]
        hh = tt + xb
        g = jax.nn.sigmoid(
            jnp.dot(xbf, wg_ref[0].astype(jnp.bfloat16),
                    preferred_element_type=jnp.float32) + bg_ref[0])
        out_ref[...] = g * hh + (1.0 - g) * xb


def _grouped_mlp(xp, be, W1, b1, W2, b2, Wg, bg, interpret=False):
    nb = xp.shape[0] // BM
    grid_spec = pltpu.PrefetchScalarGridSpec(
        num_scalar_prefetch=1,
        grid=(nb, KF),
        in_specs=[
            pl.BlockSpec((BM, DIM), lambda b, k, be: (b, 0)),
            pl.BlockSpec((1, DIM, BK), lambda b, k, be: (be[b], 0, k)),
            pl.BlockSpec((1, 1, BK), lambda b, k, be: (be[b], 0, k)),
            pl.BlockSpec((1, BK, DIM), lambda b, k, be: (be[b], k, 0)),
            pl.BlockSpec((1, 1, DIM), lambda b, k, be: (be[b], 0, 0)),
            pl.BlockSpec((1, DIM, DIM), lambda b, k, be: (be[b], 0, 0)),
            pl.BlockSpec((1, 1, DIM), lambda b, k, be: (be[b], 0, 0)),
        ],
        out_specs=pl.BlockSpec((BM, DIM), lambda b, k, be: (b, 0)),
        scratch_shapes=[pltpu.VMEM((BM, DIM), jnp.float32)],
    )
    return pl.pallas_call(
        _gated_mlp_block,
        grid_spec=grid_spec,
        out_shape=jax.ShapeDtypeStruct((xp.shape[0], DIM), jnp.float32),
        compiler_params=pltpu.CompilerParams(
            dimension_semantics=("arbitrary", "arbitrary")),
        interpret=interpret,
    )(be, xp, W1, b1, W2, b2, Wg, bg)


def kernel(x, domain_ids, sW1, sb1, sW2, sb2, sWg, sbg,
           eW1, eb1, eW2, eb2, eWg, ebg, interpret=False):
    shared_be = jnp.zeros((N // BM,), jnp.int32)
    shared_out = _grouped_mlp(x, shared_be,
                              sW1.reshape(1, DIM, FFN),
                              sb1.reshape(1, 1, FFN),
                              sW2.reshape(1, FFN, DIM),
                              sb2.reshape(1, 1, DIM),
                              sWg.reshape(1, DIM, DIM),
                              sbg.reshape(1, 1, DIM),
                              interpret=interpret)

    d = domain_ids.astype(jnp.int32)
    onehot = (d[:, None] == jnp.arange(E, dtype=jnp.int32)[None, :])
    onehot = onehot.astype(jnp.int32)
    rank = jnp.cumsum(onehot, axis=0) - onehot
    rank = jnp.take_along_axis(rank, d[:, None], axis=1)[:, 0]
    counts = jnp.sum(onehot, axis=0)
    padded = ((counts + BM - 1) // BM) * BM
    cum_padded = jnp.cumsum(padded)
    poff = cum_padded - padded
    pos_tok = poff[d] + rank

    xp = jnp.zeros((CAP, DIM), x.dtype).at[pos_tok].set(
        x, unique_indices=True)

    be = jnp.searchsorted(
        cum_padded, jnp.arange(NBLK, dtype=jnp.int32) * BM,
        side="right").astype(jnp.int32)
    be = jnp.minimum(be, E - 1)

    yp = _grouped_mlp(xp, be,
                      eW1, eb1.reshape(E, 1, FFN), eW2,
                      eb2.reshape(E, 1, DIM), eWg, ebg.reshape(E, 1, DIM),
                      interpret=interpret)

    return shared_out + yp[pos_tok]


# speedup vs baseline: 1.0271x; 1.0271x over previous
import jax
import jax.numpy as jnp
from jax.experimental import pallas as pl
from jax.experimental.pallas import tpu as pltpu

DIM = 1024
FFN = 4096
E = 8
N = 4096

BM = 512
BK = 1024
KF = FFN // BK
NBLK = N // BM + E
CAP = NBLK * BM

_INV_SQRT2 = 0.7071067811865476


def _gated_mlp_block(be_ref, nbu_ref, x_ref, w1_ref, b1_ref, w2_ref, b2_ref,
                     wg_ref, bg_ref, out_ref, acc_ref):
    b = pl.program_id(0)
    k = pl.program_id(1)

    @pl.when(b < nbu_ref[0])
    def _():
        xb = x_ref[...]
        xbf = xb.astype(jnp.bfloat16)
        h = jnp.dot(xbf, w1_ref[0].astype(jnp.bfloat16),
                    preferred_element_type=jnp.float32)
        h = h + b1_ref[0]
        h = 0.5 * h * (1.0 + jax.lax.erf(h * _INV_SQRT2))
        t = jnp.dot(h.astype(jnp.bfloat16), w2_ref[0].astype(jnp.bfloat16),
                    preferred_element_type=jnp.float32)

        @pl.when(k == 0)
        def _():
            acc_ref[...] = t

        @pl.when(k > 0)
        def _():
            acc_ref[...] = acc_ref[...] + t

        @pl.when(k == KF - 1)
        def _():
            tt = acc_ref[...] + b2_ref[0]
            hh = tt + xb
            g = jax.nn.sigmoid(
                jnp.dot(xbf, wg_ref[0].astype(jnp.bfloat16),
                        preferred_element_type=jnp.float32) + bg_ref[0])
            out_ref[...] = g * hh + (1.0 - g) * xb


def _snake(b, k):
    return jnp.where(b % 2 == 0, k, KF - 1 - k)


def _grouped_mlp(xp, be, nbu, W1, b1, W2, b2, Wg, bg, interpret=False):
    nb = xp.shape[0] // BM

    def xmap(b, k, be, nbu):
        return (jnp.minimum(b, nbu[0] - 1), 0)

    def w1map(b, k, be, nbu):
        return (be[b], 0, _snake(b, k))

    def b1map(b, k, be, nbu):
        return (be[b], 0, _snake(b, k))

    def w2map(b, k, be, nbu):
        return (be[b], _snake(b, k), 0)

    def cmap(b, k, be, nbu):
        return (be[b], 0, 0)

    def omap(b, k, be, nbu):
        return (b, 0)

    grid_spec = pltpu.PrefetchScalarGridSpec(
        num_scalar_prefetch=2,
        grid=(nb, KF),
        in_specs=[
            pl.BlockSpec((BM, DIM), xmap),
            pl.BlockSpec((1, DIM, BK), w1map),
            pl.BlockSpec((1, 1, BK), b1map),
            pl.BlockSpec((1, BK, DIM), w2map),
            pl.BlockSpec((1, 1, DIM), cmap),
            pl.BlockSpec((1, DIM, DIM), cmap),
            pl.BlockSpec((1, 1, DIM), cmap),
        ],
        out_specs=pl.BlockSpec((BM, DIM), omap),
        scratch_shapes=[pltpu.VMEM((BM, DIM), jnp.float32)],
    )
    return pl.pallas_call(
        _gated_mlp_block,
        grid_spec=grid_spec,
        out_shape=jax.ShapeDtypeStruct((xp.shape[0], DIM), jnp.float32),
        compiler_params=pltpu.CompilerParams(
            dimension_semantics=("arbitrary", "arbitrary")),
        interpret=interpret,
    )(be, nbu, xp, W1, b1, W2, b2, Wg, bg)


def kernel(x, domain_ids, sW1, sb1, sW2, sb2, sWg, sbg,
           eW1, eb1, eW2, eb2, eWg, ebg, interpret=False):
    shared_be = jnp.zeros((N // BM,), jnp.int32)
    shared_nbu = jnp.full((1,), N // BM, jnp.int32)
    shared_out = _grouped_mlp(x, shared_be, shared_nbu,
                              sW1.reshape(1, DIM, FFN),
                              sb1.reshape(1, 1, FFN),
                              sW2.reshape(1, FFN, DIM),
                              sb2.reshape(1, 1, DIM),
                              sWg.reshape(1, DIM, DIM),
                              sbg.reshape(1, 1, DIM),
                              interpret=interpret)

    d = domain_ids.astype(jnp.int32)
    onehot = (d[:, None] == jnp.arange(E, dtype=jnp.int32)[None, :])
    onehot = onehot.astype(jnp.int32)
    rank = jnp.cumsum(onehot, axis=0) - onehot
    rank = jnp.take_along_axis(rank, d[:, None], axis=1)[:, 0]
    counts = jnp.sum(onehot, axis=0)
    padded = ((counts + BM - 1) // BM) * BM
    cum_padded = jnp.cumsum(padded)
    poff = cum_padded - padded
    pos_tok = poff[d] + rank

    xp = jnp.zeros((CAP, DIM), x.dtype).at[pos_tok].set(
        x, unique_indices=True)

    nb_used = cum_padded[E - 1] // BM
    be = jnp.searchsorted(
        cum_padded, jnp.arange(NBLK, dtype=jnp.int32) * BM,
        side="right").astype(jnp.int32)
    be_last = jnp.minimum(be, E - 1)[jnp.maximum(nb_used - 1, 0)]
    be = jnp.where(jnp.arange(NBLK) < nb_used, jnp.minimum(be, E - 1),
                   be_last)
    nbu = nb_used.reshape(1).astype(jnp.int32)

    yp = _grouped_mlp(xp, be, nbu,
                      eW1, eb1.reshape(E, 1, FFN), eW2,
                      eb2.reshape(E, 1, DIM), eWg, ebg.reshape(E, 1, DIM),
                      interpret=interpret)

    return shared_out + yp[pos_tok]
